# Initial kernel scaffold; baseline (speedup 1.0000x reference)
import jax
import jax.numpy as jnp
from jax.experimental import pallas as pl


def kernel(video, audio):
    raise NotImplementedError("write your pallas kernel here")


# speedup vs baseline: 1.5653x; 1.5653x over previous
import functools

import jax
import jax.numpy as jnp
import numpy as np
from jax import lax
from jax.experimental import pallas as pl
from jax.experimental.pallas import tpu as pltpu
from jax.experimental.pallas import tpu_sc as plsc

_B, _T, _L = 16, 1024, 256
_DV, _DA = 768, 128
_NC, _NS = 2, 16
_NW = _NC * _NS
_RPW = _B * _L // _NW


_U32 = np.uint64(0xFFFFFFFF)


def _threefry2x32(k1: np.uint32, k2: np.uint32, x0: np.ndarray, x1: np.ndarray):
    x0 = x0.astype(np.uint64)
    x1 = x1.astype(np.uint64)
    ks = [np.uint64(k1), np.uint64(k2), np.uint64(0)]
    ks[2] = (ks[0] ^ ks[1] ^ np.uint64(0x1BD11BDA)) & _U32
    x0 = (x0 + ks[0]) & _U32
    x1 = (x1 + ks[1]) & _U32
    rot0, rot1 = (13, 15, 26, 6), (17, 29, 16, 24)

    def rounds(x0, x1, rots):
        for r in rots:
            x0 = (x0 + x1) & _U32
            x1 = ((x1 << np.uint64(r)) | (x1 >> np.uint64(32 - r))) & _U32
            x1 = x0 ^ x1
        return x0, x1

    for i, (rots, a, b) in enumerate(
        [(rot0, 1, 2), (rot1, 2, 0), (rot0, 0, 1), (rot1, 1, 2), (rot0, 2, 0)]):
        x0, x1 = rounds(x0, x1, rots)
        x0 = (x0 + ks[a]) & _U32
        x1 = (x1 + ks[b] + np.uint64(i + 1)) & _U32
    return x0.astype(np.uint32), x1.astype(np.uint32)


def _split(key: np.ndarray, n: int) -> np.ndarray:
    b1, b2 = _threefry2x32(key[0], key[1],
                           np.zeros(n, np.uint32), np.arange(n, dtype=np.uint32))
    return np.stack([b1, b2], axis=1)


def _random_bits32(key: np.ndarray, n: int) -> np.ndarray:
    b1, b2 = _threefry2x32(key[0], key[1],
                           np.zeros(n, np.uint32), np.arange(n, dtype=np.uint32))
    return b1 ^ b2


def _bootstrap_flat_indices() -> np.ndarray:
    key = np.array([0, 42], np.uint32)
    keys = _split(key, _B)
    idx = np.zeros((_B, _L), np.int32)
    for b in range(_B):
        sub = _split(keys[b], 2)
        lower = _random_bits32(sub[1], _L)
        idx[b] = np.sort((lower % np.uint32(_T)).astype(np.int32))
    flat = idx + (np.arange(_B, dtype=np.int32) * _T)[:, None]
    return flat.reshape(-1)


_FLAT_IDX = _bootstrap_flat_indices()


@functools.partial(
    pl.kernel,
    out_type=(
        jax.ShapeDtypeStruct((_B * _L, _DV), jnp.float32),
        jax.ShapeDtypeStruct((_B * _L, _DA), jnp.float32),
    ),
    mesh=plsc.VectorSubcoreMesh(core_axis_name="c", subcore_axis_name="s"),
    scratch_types=[
        pltpu.VMEM((_RPW,), jnp.int32),
        pltpu.VMEM((_RPW, _DV), jnp.float32),
        pltpu.VMEM((_RPW, _DA), jnp.float32),
        pltpu.SemaphoreType.DMA,
        pltpu.SemaphoreType.DMA,
    ],
)
def _gather_frames(video_hbm, audio_hbm, idx_hbm, vout_hbm, aout_hbm,
                   idx_v, vrows, arows, vsem, asem):
    wid = lax.axis_index("s") * _NC + lax.axis_index("c")
    base = wid * _RPW
    pltpu.sync_copy(idx_hbm.at[pl.ds(base, _RPW)], idx_v)
    vcopy = pltpu.async_copy(video_hbm.at[idx_v], vrows, vsem)
    acopy = pltpu.async_copy(audio_hbm.at[idx_v], arows, asem)
    vcopy.wait()
    pltpu.sync_copy(vrows, vout_hbm.at[pl.ds(base, _RPW)])
    acopy.wait()
    pltpu.sync_copy(arows, aout_hbm.at[pl.ds(base, _RPW)])


def kernel(video, audio):
    v2 = video.reshape(_B * _T, _DV)
    a2 = audio.reshape(_B * _T, _DA)
    flat_idx = jnp.asarray(_FLAT_IDX)
    vout, aout = _gather_frames(v2, a2, flat_idx)
    return vout.reshape(_B, _L, _DV), aout.reshape(_B, _L, _DA)
